# baseline (speedup 1.0000x reference)
import jax
import jax.numpy as jnp
from jax.experimental import pallas as pl


def _gatv2(x, src, dst, ea, Wl, Wr, We, att, bias, heads, out_ch):
    n = x.shape[0]
    e = ea.shape[0]
    xl = (x @ Wl).reshape(n, heads, out_ch)
    xr = (x @ Wr).reshape(n, heads, out_ch)
    em = (ea @ We).reshape(e, heads, out_ch)
    m = xl[src] + xr[dst] + em
    m = jax.nn.leaky_relu(m, 0.2)
    alpha = jnp.sum(m * att[None, :, :], axis=-1)
    amax = jax.ops.segment_max(alpha, dst, num_segments=n)
    ex = jnp.exp(alpha - amax[dst])
    den = jax.ops.segment_sum(ex, dst, num_segments=n)
    alpha = ex / (den[dst] + 1e-16)
    out = jax.ops.segment_sum(xl[src] * alpha[..., None], dst, num_segments=n)
    out = out.reshape(n, heads * out_ch)
    return out + bias


def _tail_kernel(g_ref, w_ref, b_ref, o_ref):
    o_ref[...] = g_ref[...] @ w_ref[...] + b_ref[...]


def kernel(x, edge_index, edge_attr, batch, W0, b0, Wl1, Wr1, We1, att1, b1,
           Wl2, Wr2, We2, att2, b2, Wl3, Wr3, We3, att3, b3, W4, b4):
    n = x.shape[0]
    ng = 256
    src = edge_index[0]
    dst = edge_index[1]
    s = jax.ops.segment_sum(edge_attr, dst, num_segments=n)
    c = jax.ops.segment_sum(jnp.ones((edge_attr.shape[0],), edge_attr.dtype), dst, num_segments=n)
    loop_attr = s / jnp.maximum(c, 1.0)[:, None]
    ar = jnp.arange(n, dtype=src.dtype)
    src_a = jnp.concatenate([src, ar])
    dst_a = jnp.concatenate([dst, ar])
    ea = jnp.concatenate([edge_attr, loop_attr], axis=0)
    ohe = x[:, :1] @ W0 + b0
    h = jnp.concatenate([ohe, x[:, 1:]], axis=1)
    h = jnp.tanh(_gatv2(h, src_a, dst_a, ea, Wl1, Wr1, We1, att1, b1, 8, 32))
    h = jnp.tanh(_gatv2(h, src_a, dst_a, ea, Wl2, Wr2, We2, att2, b2, 8, 16))
    h = jnp.tanh(_gatv2(h, src_a, dst_a, ea, Wl3, Wr3, We3, att3, b3, 1, 8))
    sums = jax.ops.segment_sum(h, batch, num_segments=ng)
    cnts = jax.ops.segment_sum(jnp.ones((n,), h.dtype), batch, num_segments=ng)
    g = sums / jnp.maximum(cnts, 1.0)[:, None]
    out = pl.pallas_call(
        _tail_kernel,
        out_shape=jax.ShapeDtypeStruct((ng, 1), jnp.float32),
    )(g, W4, b4)
    return out


# speedup vs baseline: 12.0550x; 12.0550x over previous
import functools

import jax
import jax.numpy as jnp
from jax import lax
from jax.experimental import pallas as pl
from jax.experimental.pallas import tpu as pltpu
from jax.experimental.pallas import tpu_sc as plsc

N = 50000
E = 800000
NG = 256
NC, NS, L = 2, 16, 16
NT = NC * NS
N_T = 50048
CHUNK = 128

_mesh = plsc.VectorSubcoreMesh(core_axis_name="c", subcore_axis_name="s",
                               num_cores=NC, num_subcores=NS)
_sc_params = pltpu.CompilerParams(use_tc_tiling_on_sc=False,
                                  needs_layout_passes=False)


def _iota16():
    return lax.broadcasted_iota(jnp.int32, (L,), 0)


def _zero_rows(zbuf, n_rows):
    zv = jnp.zeros((L,), jnp.float32)

    def body(i, _):
        zbuf[i, :] = zv
        return 0

    lax.fori_loop(0, n_rows, body, 0)


E_P = 802816
_K0_CHUNKS = E_P // (NT * CHUNK)
_ZROWS = 782


def _k0_body(dst_hbm, ea_hbm, out_hbm, dstb, eab, cb, zbuf, sp):
    c = lax.axis_index("c")
    s = lax.axis_index("s")
    tid = s * NC + c

    _zero_rows(zbuf, _ZROWS)
    for j in range(4):
        pltpu.sync_copy(zbuf, sp.at[pl.ds((s * 4 + j) * _ZROWS, _ZROWS)])
    _zero_rows(cb, CHUNK)
    ones = jnp.ones((L,), jnp.float32)
    col1 = jnp.ones((L,), jnp.int32)
    for g in range(CHUNK // L):
        plsc.store_scatter(cb, [_iota16() + g * L, col1], ones)
    plsc.subcore_barrier()

    def chunk_body(i, _):
        base = tid * (_K0_CHUNKS * CHUNK) + i * CHUNK
        pltpu.sync_copy(dst_hbm.at[pl.ds(base, CHUNK)], dstb)
        pltpu.sync_copy(ea_hbm.at[pl.ds(base, CHUNK)], eab)
        col0 = jnp.zeros((L,), jnp.int32)
        for g in range(CHUNK // L):
            eav = eab[pl.ds(g * L, L)]
            plsc.store_scatter(cb, [_iota16() + g * L, col0], eav)
        pltpu.sync_copy(cb, sp.at[dstb], add=True)
        return 0

    lax.fori_loop(0, _K0_CHUNKS, chunk_body, 0)
    plsc.subcore_barrier()

    @pl.when(s == 0)
    def _():
        pltpu.sync_copy(sp, out_hbm.at[c])


_k0 = functools.partial(
    pl.kernel,
    out_type=jax.ShapeDtypeStruct((NC, N_T, 16), jnp.float32),
    mesh=_mesh,
    compiler_params=_sc_params,
    scratch_types=[
        pltpu.VMEM((CHUNK,), jnp.int32),
        pltpu.VMEM((CHUNK,), jnp.float32),
        pltpu.VMEM((CHUNK, 16), jnp.float32),
        pltpu.VMEM((_ZROWS, 16), jnp.float32),
        pltpu.VMEM_SHARED((N_T, 16), jnp.float32),
    ],
)(_k0_body)


def _loop_attr_sc(dst32, ea_flat):
    dst_p = jnp.full((E_P,), N, jnp.int32).at[:E].set(dst32)
    ea_p = jnp.zeros((E_P,), jnp.float32).at[:E].set(ea_flat)
    parts = _k0(dst_p, ea_p)
    s = parts[0, :N, 0] + parts[1, :N, 0]
    cnt = parts[0, :N, 1] + parts[1, :N, 1]
    return s / jnp.maximum(cnt, 1.0)


EA = E + N
EA_P = 851968
HP = 16
CA = 64
_A_CHUNKS = EA_P // (NT * CA)
_AZ = 92


def _extract_col(metab, col, outb, nrows):
    cv = jnp.full((L,), col, jnp.int32)
    for g in range(nrows // L):
        v = plsc.load_gather(metab, [_iota16() + g * L, cv])
        outb[pl.ds(g * L, L)] = v


def _make_passA(C, H):
    D = C // H

    def body(meta_hbm, xl_hbm, xr_hbm, wv_hbm, att_hbm, ex_hbm, den_hbm,
             metab, srcb, dstb, xlb, xrb, exb, wvb, attb, zbuf, den_sp,
             sem1, sem2):
        c = lax.axis_index("c")
        s = lax.axis_index("s")
        tid = s * NC + c
        pltpu.sync_copy(wv_hbm, wvb)
        pltpu.sync_copy(att_hbm, attb)
        _zero_rows(zbuf, _AZ)
        nz = (N_T // NS) // _AZ
        def zbody(j, _):
            pltpu.sync_copy(zbuf, den_sp.at[pl.ds(s * (N_T // NS) + j * _AZ, _AZ)])
            return 0
        lax.fori_loop(0, nz, zbody, 0)
        _zero_rows(exb, CA)
        plsc.subcore_barrier()

        zi = jnp.zeros((L,), jnp.int32)

        def chunk_body(i, _):
            base = tid * (_A_CHUNKS * CA) + i * CA
            pltpu.sync_copy(meta_hbm.at[pl.ds(base, CA)], metab)
            _extract_col(metab, 0, srcb, CA)
            _extract_col(metab, 1, dstb, CA)
            a1 = pltpu.async_copy(xl_hbm.at[srcb], xlb, sem1)
            a2 = pltpu.async_copy(xr_hbm.at[dstb], xrb, sem2)
            a1.wait()
            a2.wait()
            c2 = jnp.full((L,), 2, jnp.int32)
            for g in range(CA // L):
                rows = _iota16() + g * L
                eav = plsc.bitcast(plsc.load_gather(metab, [rows, c2]),
                                   jnp.float32)
                for h in range(H):
                    def dbody(d, acc, _h=h, _rows=rows, _eav=eav):
                        cc = _h * D + d
                        ccv = zi + cc
                        xlv = plsc.load_gather(xlb, [_rows, ccv])
                        xrv = plsc.load_gather(xrb, [_rows, ccv])
                        av = xlv + xrv + _eav * wvb[cc, :]
                        lv = jnp.maximum(av, av * jnp.float32(0.2))
                        return acc + lv * attb[cc, :]

                    acc = plsc.parallel_loop(
                        0, D, unroll=8,
                        carry=jnp.zeros((L,), jnp.float32))(dbody)
                    plsc.store_scatter(exb, [rows, zi + h], jnp.exp(acc))
            pltpu.sync_copy(exb, ex_hbm.at[pl.ds(base, CA)])
            pltpu.sync_copy(exb, den_sp.at[dstb], add=True)
            return 0

        lax.fori_loop(0, _A_CHUNKS, chunk_body, 0)
        plsc.subcore_barrier()

        @pl.when(s == 0)
        def _():
            pltpu.sync_copy(den_sp, den_hbm.at[c])

    return functools.partial(
        pl.kernel,
        out_type=(jax.ShapeDtypeStruct((EA_P, HP), jnp.float32),
                  jax.ShapeDtypeStruct((NC, N_T, HP), jnp.float32)),
        mesh=_mesh,
        compiler_params=_sc_params,
        scratch_types=[
            pltpu.VMEM((CA, 4), jnp.int32),
            pltpu.VMEM((CA,), jnp.int32),
            pltpu.VMEM((CA,), jnp.int32),
            pltpu.VMEM((CA, C), jnp.float32),
            pltpu.VMEM((CA, C), jnp.float32),
            pltpu.VMEM((CA, HP), jnp.float32),
            pltpu.VMEM((C, 16), jnp.float32),
            pltpu.VMEM((C, 16), jnp.float32),
            pltpu.VMEM((_AZ, 16), jnp.float32),
            pltpu.VMEM_SHARED((N_T, HP), jnp.float32),
            pltpu.SemaphoreType.DMA,
            pltpu.SemaphoreType.DMA,
        ],
    )(body)


_passA = {256: _make_passA(256, 8), 128: _make_passA(128, 8),
          16: _make_passA(16, 1)}


_A2_CHUNKS = EA_P // (NT * CHUNK)


def _a2_body(meta_hbm, ex_hbm, den0_hbm, den1_hbm, w_hbm,
             metab, dstb, exb2, d0b, d1b, wb, sem1, sem2, sem3):
    c = lax.axis_index("c")
    s = lax.axis_index("s")
    tid = s * NC + c
    eps = jnp.float32(1e-16)

    def chunk_body(i, _):
        base = tid * (_A2_CHUNKS * CHUNK) + i * CHUNK
        pltpu.sync_copy(meta_hbm.at[pl.ds(base, CHUNK)], metab)
        _extract_col(metab, 1, dstb, CHUNK)
        a0 = pltpu.async_copy(ex_hbm.at[pl.ds(base, CHUNK)], exb2, sem1)
        a1 = pltpu.async_copy(den0_hbm.at[dstb], d0b, sem2)
        a2 = pltpu.async_copy(den1_hbm.at[dstb], d1b, sem3)
        a0.wait()
        a1.wait()
        a2.wait()

        def rbody(g, _2):
            wv = exb2[g, :] / (d0b[g, :] + d1b[g, :] + eps)
            wb[g, :] = wv
            return 0

        lax.fori_loop(0, CHUNK, rbody, 0)
        pltpu.sync_copy(wb, w_hbm.at[pl.ds(base, CHUNK)])
        return 0

    lax.fori_loop(0, _A2_CHUNKS, chunk_body, 0)


_a2 = functools.partial(
    pl.kernel,
    out_type=jax.ShapeDtypeStruct((EA_P, HP), jnp.float32),
    mesh=_mesh,
    compiler_params=_sc_params,
    scratch_types=[
        pltpu.VMEM((CHUNK, 4), jnp.int32),
        pltpu.VMEM((CHUNK,), jnp.int32),
        pltpu.VMEM((CHUNK, HP), jnp.float32),
        pltpu.VMEM((CHUNK, HP), jnp.float32),
        pltpu.VMEM((CHUNK, HP), jnp.float32),
        pltpu.VMEM((CHUNK, HP), jnp.float32),
        pltpu.SemaphoreType.DMA,
        pltpu.SemaphoreType.DMA,
        pltpu.SemaphoreType.DMA,
    ],
)(_a2_body)


def _attention_weights_sc(meta, xl_t, xr_t, We_row, att_flat, C, H):
    ex, den = _passA[C](meta, xl_t, xr_t, We_row, att_flat)
    return _a2(meta, ex, den[0], den[1])


CB = 64
_TCH = EA_P // CB
_B_SEGS = 16
_B_SEG_CHUNKS = _TCH // NS // _B_SEGS


def _make_passB(C, H, RB, RB_P, CAP, r0g):
    D = C // H

    def body(meta_hbm, w_hbm, xl_hbm, out_hbm,
             metab, selsrc, seldst, seleid, idxs, idxd, idxe,
             rowsb, wb, zbufB, out_sp, sem1, sem2):
        c = lax.axis_index("c")
        s = lax.axis_index("s")
        sc_lo = r0g + c * RB

        _zero_rows_c(zbufB, 8, C)
        zrows = RB_P // NS
        def zbody(j, _):
            pltpu.sync_copy(zbufB, out_sp.at[pl.ds(s * zrows + j * 8, 8)])
            return 0
        lax.fori_loop(0, zrows // 8, zbody, 0)
        plsc.subcore_barrier()

        zi = jnp.zeros((L,), jnp.int32)
        c0v = zi
        c1v = jnp.full((L,), 1, jnp.int32)
        zf = jnp.zeros((L,), jnp.float32)

        def seg_body(q, _seg):
            def scan_body(j, off):
                k = s + NS * (q * _B_SEG_CHUNKS + j)
                base = k * CB
                pltpu.sync_copy(meta_hbm.at[pl.ds(base, CB)], metab)
                for g in range(CB // L):
                    rows = _iota16() + g * L
                    srcv = plsc.load_gather(metab, [rows, c0v])
                    dstv = plsc.load_gather(metab, [rows, c1v])
                    lm = (dstv >= sc_lo) & (dstv < sc_lo + RB)
                    dl = dstv - sc_lo
                    eidv = rows + base
                    plsc.store_compressed(selsrc.at[pl.ds(off, L)], srcv, mask=lm)
                    plsc.store_compressed(seldst.at[pl.ds(off, L)], dl, mask=lm)
                    plsc.store_compressed(seleid.at[pl.ds(off, L)], eidv, mask=lm)
                    off = off + plsc.all_reduce_population_count(lm)[0]
                return off

            off = lax.fori_loop(0, _B_SEG_CHUNKS, scan_body, jnp.int32(0))
            dumpv = jnp.full((L,), RB, jnp.int32)
            for g in range(CB // L):
                selsrc[pl.ds(off + g * L, L)] = zi
                seldst[pl.ds(off + g * L, L)] = dumpv
                seleid[pl.ds(off + g * L, L)] = zi
            nb = (off + CB - 1) // CB

            def bbody(b, _):
                for g in range(CB // L):
                    idxs[pl.ds(g * L, L)] = selsrc[pl.ds(b * CB + g * L, L)]
                    idxd[pl.ds(g * L, L)] = seldst[pl.ds(b * CB + g * L, L)]
                    idxe[pl.ds(g * L, L)] = seleid[pl.ds(b * CB + g * L, L)]
                a1 = pltpu.async_copy(xl_hbm.at[idxs], rowsb, sem1)
                a2 = pltpu.async_copy(w_hbm.at[idxe], wb, sem2)
                a1.wait()
                a2.wait()
                for g in range(CB // L):
                    rows = _iota16() + g * L
                    for h in range(H):
                        whv = plsc.load_gather(wb, [rows, zi + h])

                        def dbody(d, dummy, _h=h, _rows=rows, _whv=whv):
                            cc = _h * D + d
                            ccv = zi + cc
                            rv = plsc.load_gather(rowsb, [_rows, ccv]) * _whv
                            plsc.store_scatter(rowsb, [_rows, ccv], rv)
                            return dummy

                        plsc.parallel_loop(0, D, unroll=8,
                                           carry=jnp.int32(0))(dbody)
                pltpu.sync_copy(rowsb, out_sp.at[idxd], add=True)
                return 0

            lax.fori_loop(0, nb, bbody, 0)
            return 0

        lax.fori_loop(0, _B_SEGS, seg_body, 0)
        plsc.subcore_barrier()

        @pl.when(s == 0)
        def _():
            pltpu.sync_copy(out_sp.at[pl.ds(0, RB)],
                            out_hbm.at[pl.ds(c * RB, RB)])

    return functools.partial(
        pl.kernel,
        out_type=jax.ShapeDtypeStruct((NC * RB, C), jnp.float32),
        mesh=_mesh,
        compiler_params=_sc_params,
        scratch_types=[
            pltpu.VMEM((CB, 4), jnp.int32),
            pltpu.VMEM((CAP,), jnp.int32),
            pltpu.VMEM((CAP,), jnp.int32),
            pltpu.VMEM((CAP,), jnp.int32),
            pltpu.VMEM((CB,), jnp.int32),
            pltpu.VMEM((CB,), jnp.int32),
            pltpu.VMEM((CB,), jnp.int32),
            pltpu.VMEM((CB, C), jnp.float32),
            pltpu.VMEM((CB, HP), jnp.float32),
            pltpu.VMEM((8, C), jnp.float32),
            pltpu.VMEM_SHARED((RB_P, C), jnp.float32),
            pltpu.SemaphoreType.DMA,
            pltpu.SemaphoreType.DMA,
        ],
    )(body)


def _zero_rows_c(zbuf, n_rows, C):
    zv = jnp.zeros((L,), jnp.float32)

    def body(i, _):
        for j in range(C // L):
            zbuf[i, pl.ds(j * L, L)] = zv
        return 0

    lax.fori_loop(0, n_rows, body, 0)


_B_GEOM = {256: (8, 6016, 6144, 1536, 5),
           128: (8, 12544, 12800, 2560, 2),
           16: (1, 25088, 25344, 3456, 1)}
_passB = {}
for _C, (_H, _RB, _RBP, _CAP, _P) in _B_GEOM.items():
    _passB[_C] = [_make_passB(_C, _H, _RB, _RBP, _CAP, _p * NC * _RB)
                  for _p in range(_P)]


def _aggregate_sc(meta, w, xl_t, C):
    _, RB, _, _, P = _B_GEOM[C]
    pieces = [_passB[C][p](meta, w, xl_t) for p in range(P)]
    return jnp.concatenate(pieces, axis=0)


def _gatv2_sc(x, meta, src, dst, Wl, Wr, We, att, bias, heads, out_ch):
    n = x.shape[0]
    C = heads * out_ch
    xl = x @ Wl
    xr = x @ Wr
    xl_t = jnp.zeros((N_T, C), jnp.float32).at[:n].set(xl)
    xr_t = jnp.zeros((N_T, C), jnp.float32).at[:n].set(xr)
    we_rep = jnp.tile(We[0][:, None], (1, 16))
    att_rep = jnp.tile(att.reshape(-1)[:, None], (1, 16))
    w = _attention_weights_sc(meta, xl_t, xr_t, we_rep, att_rep, C, heads)
    out = _aggregate_sc(meta, w, xl_t, C)[:n]
    return out + bias


def _tail_kernel(g_ref, w_ref, b_ref, o_ref):
    o_ref[...] = g_ref[...] @ w_ref[...] + b_ref[...]


def kernel(x, edge_index, edge_attr, batch, W0, b0, Wl1, Wr1, We1, att1, b1,
           Wl2, Wr2, We2, att2, b2, Wl3, Wr3, We3, att3, b3, W4, b4):
    n = x.shape[0]
    src = edge_index[0].astype(jnp.int32)
    dst = edge_index[1].astype(jnp.int32)
    loop_attr = _loop_attr_sc(dst, edge_attr[:, 0])
    ar = jnp.arange(n, dtype=jnp.int32)
    src_a = jnp.concatenate([src, ar])
    dst_a = jnp.concatenate([dst, ar])
    ea_all = jnp.concatenate([edge_attr[:, 0], loop_attr])
    src_p = jnp.zeros((EA_P,), jnp.int32).at[:EA].set(src_a)
    dst_p = jnp.full((EA_P,), N, jnp.int32).at[:EA].set(dst_a)
    ea_p = jnp.zeros((EA_P,), jnp.float32).at[:EA].set(ea_all)
    meta = jnp.stack(
        [src_p, dst_p, jax.lax.bitcast_convert_type(ea_p, jnp.int32),
         jnp.zeros((EA_P,), jnp.int32)], axis=1)
    ohe = x[:, :1] @ W0 + b0
    h = jnp.concatenate([ohe, x[:, 1:]], axis=1)
    h = jnp.tanh(_gatv2_sc(h, meta, src_a, dst_a, Wl1, Wr1, We1, att1, b1, 8, 32))
    h = jnp.tanh(_gatv2_sc(h, meta, src_a, dst_a, Wl2, Wr2, We2, att2, b2, 8, 16))
    Wl3p = jnp.pad(Wl3, ((0, 0), (0, 8)))
    Wr3p = jnp.pad(Wr3, ((0, 0), (0, 8)))
    We3p = jnp.pad(We3, ((0, 0), (0, 8)))
    att3p = jnp.pad(att3, ((0, 0), (0, 8)))
    h = jnp.tanh(_gatv2_sc(h, meta, src_a, dst_a, Wl3p, Wr3p, We3p, att3p,
                           jnp.pad(b3, (0, 8)), 1, 16)[:, :8])
    sums = jax.ops.segment_sum(h, batch, num_segments=NG)
    cnts = jax.ops.segment_sum(jnp.ones((n,), h.dtype), batch, num_segments=NG)
    g = sums / jnp.maximum(cnts, 1.0)[:, None]
    out = pl.pallas_call(
        _tail_kernel,
        out_shape=jax.ShapeDtypeStruct((NG, 1), jnp.float32),
    )(g, W4, b4)
    return out


# speedup vs baseline: 12.1431x; 1.0073x over previous
import functools

import jax
import jax.numpy as jnp
from jax import lax
from jax.experimental import pallas as pl
from jax.experimental.pallas import tpu as pltpu
from jax.experimental.pallas import tpu_sc as plsc

N = 50000
E = 800000
NG = 256
NC, NS, L = 2, 16, 16
NT = NC * NS
N_T = 50048
CHUNK = 128

_mesh = plsc.VectorSubcoreMesh(core_axis_name="c", subcore_axis_name="s",
                               num_cores=NC, num_subcores=NS)
_sc_params = pltpu.CompilerParams(use_tc_tiling_on_sc=False,
                                  needs_layout_passes=False)


def _iota16():
    return lax.broadcasted_iota(jnp.int32, (L,), 0)


def _zero_rows(zbuf, n_rows):
    zv = jnp.zeros((L,), jnp.float32)

    def body(i, _):
        zbuf[i, :] = zv
        return 0

    lax.fori_loop(0, n_rows, body, 0)


E_P = 802816
_K0_CHUNKS = E_P // (NT * CHUNK)
_ZROWS = 782


def _k0_body(dst_hbm, ea_hbm, out_hbm, dstb, eab, cb, zbuf, sp):
    c = lax.axis_index("c")
    s = lax.axis_index("s")
    tid = s * NC + c

    _zero_rows(zbuf, _ZROWS)
    for j in range(4):
        pltpu.sync_copy(zbuf, sp.at[pl.ds((s * 4 + j) * _ZROWS, _ZROWS)])
    _zero_rows(cb, CHUNK)
    ones = jnp.ones((L,), jnp.float32)
    col1 = jnp.ones((L,), jnp.int32)
    for g in range(CHUNK // L):
        plsc.store_scatter(cb, [_iota16() + g * L, col1], ones)
    plsc.subcore_barrier()

    def chunk_body(i, _):
        base = tid * (_K0_CHUNKS * CHUNK) + i * CHUNK
        pltpu.sync_copy(dst_hbm.at[pl.ds(base, CHUNK)], dstb)
        pltpu.sync_copy(ea_hbm.at[pl.ds(base, CHUNK)], eab)
        col0 = jnp.zeros((L,), jnp.int32)
        for g in range(CHUNK // L):
            eav = eab[pl.ds(g * L, L)]
            plsc.store_scatter(cb, [_iota16() + g * L, col0], eav)
        pltpu.sync_copy(cb, sp.at[dstb], add=True)
        return 0

    lax.fori_loop(0, _K0_CHUNKS, chunk_body, 0)
    plsc.subcore_barrier()

    @pl.when(s == 0)
    def _():
        pltpu.sync_copy(sp, out_hbm.at[c])


_k0 = functools.partial(
    pl.kernel,
    out_type=jax.ShapeDtypeStruct((NC, N_T, 16), jnp.float32),
    mesh=_mesh,
    compiler_params=_sc_params,
    scratch_types=[
        pltpu.VMEM((CHUNK,), jnp.int32),
        pltpu.VMEM((CHUNK,), jnp.float32),
        pltpu.VMEM((CHUNK, 16), jnp.float32),
        pltpu.VMEM((_ZROWS, 16), jnp.float32),
        pltpu.VMEM_SHARED((N_T, 16), jnp.float32),
    ],
)(_k0_body)


def _loop_attr_sc(dst32, ea_flat):
    dst_p = jnp.full((E_P,), N, jnp.int32).at[:E].set(dst32)
    ea_p = jnp.zeros((E_P,), jnp.float32).at[:E].set(ea_flat)
    parts = _k0(dst_p, ea_p)
    s = parts[0, :N, 0] + parts[1, :N, 0]
    cnt = parts[0, :N, 1] + parts[1, :N, 1]
    return s / jnp.maximum(cnt, 1.0)


EA = E + N
EA_P = 851968
HP = 16
CA = 64
_A_CHUNKS = EA_P // (NT * CA)
_AZ = 92


def _extract_col(metab, col, outb, nrows):
    cv = jnp.full((L,), col, jnp.int32)
    for g in range(nrows // L):
        v = plsc.load_gather(metab, [_iota16() + g * L, cv])
        outb[pl.ds(g * L, L)] = v


def _make_passA(C, H):
    D = C // H

    def body(meta_hbm, xl_hbm, xr_hbm, wv_hbm, att_hbm, ex_hbm, den_hbm,
             metab, srcb, dstb, xlb, xrb, exb, wvb, attb, zbuf, den_sp,
             sem1, sem2):
        c = lax.axis_index("c")
        s = lax.axis_index("s")
        tid = s * NC + c
        pltpu.sync_copy(wv_hbm, wvb)
        pltpu.sync_copy(att_hbm, attb)
        _zero_rows(zbuf, _AZ)
        nz = (N_T // NS) // _AZ
        def zbody(j, _):
            pltpu.sync_copy(zbuf, den_sp.at[pl.ds(s * (N_T // NS) + j * _AZ, _AZ)])
            return 0
        lax.fori_loop(0, nz, zbody, 0)
        _zero_rows(exb, CA)
        plsc.subcore_barrier()

        zi = jnp.zeros((L,), jnp.int32)

        def chunk_body(i, _):
            base = tid * (_A_CHUNKS * CA) + i * CA
            pltpu.sync_copy(meta_hbm.at[pl.ds(base, CA)], metab)
            _extract_col(metab, 0, srcb, CA)
            _extract_col(metab, 1, dstb, CA)
            a1 = pltpu.async_copy(xl_hbm.at[srcb], xlb, sem1)
            a2 = pltpu.async_copy(xr_hbm.at[dstb], xrb, sem2)
            a1.wait()
            a2.wait()
            c2 = jnp.full((L,), 2, jnp.int32)
            for g in range(CA // L):
                rows = _iota16() + g * L
                eav = plsc.bitcast(plsc.load_gather(metab, [rows, c2]),
                                   jnp.float32)
                for h in range(H):
                    def dbody(d, acc, _h=h, _rows=rows, _eav=eav):
                        cc = _h * D + d
                        ccv = zi + cc
                        xlv = plsc.load_gather(xlb, [_rows, ccv])
                        xrv = plsc.load_gather(xrb, [_rows, ccv])
                        av = xlv + xrv + _eav * wvb[cc, :]
                        lv = jnp.maximum(av, av * jnp.float32(0.2))
                        return acc + lv * attb[cc, :]

                    acc = plsc.parallel_loop(
                        0, D, unroll=8,
                        carry=jnp.zeros((L,), jnp.float32))(dbody)
                    plsc.store_scatter(exb, [rows, zi + h], jnp.exp(acc))
            pltpu.sync_copy(exb, ex_hbm.at[pl.ds(base, CA)])
            pltpu.sync_copy(exb, den_sp.at[dstb], add=True)
            return 0

        lax.fori_loop(0, _A_CHUNKS, chunk_body, 0)
        plsc.subcore_barrier()

        @pl.when(s == 0)
        def _():
            pltpu.sync_copy(den_sp, den_hbm.at[c])

    return functools.partial(
        pl.kernel,
        out_type=(jax.ShapeDtypeStruct((EA_P, HP), jnp.float32),
                  jax.ShapeDtypeStruct((NC, N_T, HP), jnp.float32)),
        mesh=_mesh,
        compiler_params=_sc_params,
        scratch_types=[
            pltpu.VMEM((CA, 4), jnp.int32),
            pltpu.VMEM((CA,), jnp.int32),
            pltpu.VMEM((CA,), jnp.int32),
            pltpu.VMEM((CA, C), jnp.float32),
            pltpu.VMEM((CA, C), jnp.float32),
            pltpu.VMEM((CA, HP), jnp.float32),
            pltpu.VMEM((C, 16), jnp.float32),
            pltpu.VMEM((C, 16), jnp.float32),
            pltpu.VMEM((_AZ, 16), jnp.float32),
            pltpu.VMEM_SHARED((N_T, HP), jnp.float32),
            pltpu.SemaphoreType.DMA,
            pltpu.SemaphoreType.DMA,
        ],
    )(body)


_passA = {256: _make_passA(256, 8), 128: _make_passA(128, 8),
          16: _make_passA(16, 1)}


_A2_CHUNKS = EA_P // (NT * CHUNK)


def _a2_body(meta_hbm, ex_hbm, den0_hbm, den1_hbm, w_hbm,
             metab, dstb, exb2, d0b, d1b, wb, sem1, sem2, sem3):
    c = lax.axis_index("c")
    s = lax.axis_index("s")
    tid = s * NC + c
    eps = jnp.float32(1e-16)

    def chunk_body(i, _):
        base = tid * (_A2_CHUNKS * CHUNK) + i * CHUNK
        pltpu.sync_copy(meta_hbm.at[pl.ds(base, CHUNK)], metab)
        _extract_col(metab, 1, dstb, CHUNK)
        a0 = pltpu.async_copy(ex_hbm.at[pl.ds(base, CHUNK)], exb2, sem1)
        a1 = pltpu.async_copy(den0_hbm.at[dstb], d0b, sem2)
        a2 = pltpu.async_copy(den1_hbm.at[dstb], d1b, sem3)
        a0.wait()
        a1.wait()
        a2.wait()

        def rbody(g, _2):
            wv = exb2[g, :] / (d0b[g, :] + d1b[g, :] + eps)
            wb[g, :] = wv
            return 0

        lax.fori_loop(0, CHUNK, rbody, 0)
        pltpu.sync_copy(wb, w_hbm.at[pl.ds(base, CHUNK)])
        return 0

    lax.fori_loop(0, _A2_CHUNKS, chunk_body, 0)


_a2 = functools.partial(
    pl.kernel,
    out_type=jax.ShapeDtypeStruct((EA_P, HP), jnp.float32),
    mesh=_mesh,
    compiler_params=_sc_params,
    scratch_types=[
        pltpu.VMEM((CHUNK, 4), jnp.int32),
        pltpu.VMEM((CHUNK,), jnp.int32),
        pltpu.VMEM((CHUNK, HP), jnp.float32),
        pltpu.VMEM((CHUNK, HP), jnp.float32),
        pltpu.VMEM((CHUNK, HP), jnp.float32),
        pltpu.VMEM((CHUNK, HP), jnp.float32),
        pltpu.SemaphoreType.DMA,
        pltpu.SemaphoreType.DMA,
        pltpu.SemaphoreType.DMA,
    ],
)(_a2_body)


def _attention_weights_sc(meta, xl_t, xr_t, We_row, att_flat, C, H):
    ex, den = _passA[C](meta, xl_t, xr_t, We_row, att_flat)
    return _a2(meta, ex, den[0], den[1])


CB = 64
_TCH = EA_P // CB
_B_SEGS = 16
_B_SEG_CHUNKS = _TCH // NS // _B_SEGS


def _make_passB(C, H, RB, RB_P, CAP, r0g):
    D = C // H

    def body(meta_hbm, w_hbm, xl_hbm, out_hbm,
             metab, selsrc, seldst, seleid, idxs, idxd, idxe,
             rowsb, wb, zbufB, out_sp, sem1, sem2):
        c = lax.axis_index("c")
        s = lax.axis_index("s")
        sc_lo = r0g + c * RB

        _zero_rows_c(zbufB, 8, C)
        zrows = RB_P // NS
        def zbody(j, _):
            pltpu.sync_copy(zbufB, out_sp.at[pl.ds(s * zrows + j * 8, 8)])
            return 0
        lax.fori_loop(0, zrows // 8, zbody, 0)
        plsc.subcore_barrier()

        zi = jnp.zeros((L,), jnp.int32)
        c0v = zi
        c1v = jnp.full((L,), 1, jnp.int32)
        zf = jnp.zeros((L,), jnp.float32)

        def seg_body(q, _seg):
            def scan_body(j, off):
                k = s + NS * (q * _B_SEG_CHUNKS + j)
                base = k * CB
                pltpu.sync_copy(meta_hbm.at[pl.ds(base, CB)], metab)
                for g in range(CB // L):
                    rows = _iota16() + g * L
                    srcv = plsc.load_gather(metab, [rows, c0v])
                    dstv = plsc.load_gather(metab, [rows, c1v])
                    lm = (dstv >= sc_lo) & (dstv < sc_lo + RB)
                    dl = dstv - sc_lo
                    eidv = rows + base
                    plsc.store_compressed(selsrc.at[pl.ds(off, L)], srcv, mask=lm)
                    plsc.store_compressed(seldst.at[pl.ds(off, L)], dl, mask=lm)
                    plsc.store_compressed(seleid.at[pl.ds(off, L)], eidv, mask=lm)
                    off = off + plsc.all_reduce_population_count(lm)[0]
                return off

            off = lax.fori_loop(0, _B_SEG_CHUNKS, scan_body, jnp.int32(0))
            dumpv = jnp.full((L,), RB, jnp.int32)
            for g in range(CB // L):
                selsrc[pl.ds(off + g * L, L)] = zi
                seldst[pl.ds(off + g * L, L)] = dumpv
                seleid[pl.ds(off + g * L, L)] = zi
            nb = (off + CB - 1) // CB

            def bbody(b, _):
                for g in range(CB // L):
                    idxs[pl.ds(g * L, L)] = selsrc[pl.ds(b * CB + g * L, L)]
                    idxd[pl.ds(g * L, L)] = seldst[pl.ds(b * CB + g * L, L)]
                    idxe[pl.ds(g * L, L)] = seleid[pl.ds(b * CB + g * L, L)]
                a1 = pltpu.async_copy(xl_hbm.at[idxs], rowsb, sem1)
                a2 = pltpu.async_copy(w_hbm.at[idxe], wb, sem2)
                a1.wait()
                a2.wait()
                for g in range(CB // L):
                    rows = _iota16() + g * L
                    for h in range(H):
                        whv = plsc.load_gather(wb, [rows, zi + h])

                        def dbody(d, dummy, _h=h, _rows=rows, _whv=whv):
                            cc = _h * D + d
                            ccv = zi + cc
                            rv = plsc.load_gather(rowsb, [_rows, ccv]) * _whv
                            plsc.store_scatter(rowsb, [_rows, ccv], rv)
                            return dummy

                        plsc.parallel_loop(0, D, unroll=8,
                                           carry=jnp.int32(0))(dbody)
                pltpu.sync_copy(rowsb, out_sp.at[idxd], add=True)
                return 0

            lax.fori_loop(0, nb, bbody, 0)
            return 0

        lax.fori_loop(0, _B_SEGS, seg_body, 0)
        plsc.subcore_barrier()

        @pl.when(s == 0)
        def _():
            pltpu.sync_copy(out_sp.at[pl.ds(0, RB)],
                            out_hbm.at[pl.ds(c * RB, RB)])

    return functools.partial(
        pl.kernel,
        out_type=jax.ShapeDtypeStruct((NC * RB, C), jnp.float32),
        mesh=_mesh,
        compiler_params=_sc_params,
        scratch_types=[
            pltpu.VMEM((CB, 4), jnp.int32),
            pltpu.VMEM((CAP,), jnp.int32),
            pltpu.VMEM((CAP,), jnp.int32),
            pltpu.VMEM((CAP,), jnp.int32),
            pltpu.VMEM((CB,), jnp.int32),
            pltpu.VMEM((CB,), jnp.int32),
            pltpu.VMEM((CB,), jnp.int32),
            pltpu.VMEM((CB, C), jnp.float32),
            pltpu.VMEM((CB, HP), jnp.float32),
            pltpu.VMEM((8, C), jnp.float32),
            pltpu.VMEM_SHARED((RB_P, C), jnp.float32),
            pltpu.SemaphoreType.DMA,
            pltpu.SemaphoreType.DMA,
        ],
    )(body)


def _zero_rows_c(zbuf, n_rows, C):
    zv = jnp.zeros((L,), jnp.float32)

    def body(i, _):
        for j in range(C // L):
            zbuf[i, pl.ds(j * L, L)] = zv
        return 0

    lax.fori_loop(0, n_rows, body, 0)


_B_GEOM = {256: (8, 6016, 6144, 1536, 5),
           128: (8, 12544, 12800, 2560, 2),
           16: (1, 25088, 25344, 3456, 1)}
_passB = {}
for _C, (_H, _RB, _RBP, _CAP, _P) in _B_GEOM.items():
    _passB[_C] = [_make_passB(_C, _H, _RB, _RBP, _CAP, _p * NC * _RB)
                  for _p in range(_P)]


def _aggregate_sc(meta, w, xl_t, C):
    _, RB, _, _, P = _B_GEOM[C]
    pieces = [_passB[C][p](meta, w, xl_t) for p in range(P)]
    return jnp.concatenate(pieces, axis=0)


N_PP = 53248
_P_CHUNKS = N_PP // (NT * CHUNK)
NGP = NG + 64


def _pool_body(h_hbm, b_hbm, pool_hbm, cnt_hbm,
               hb, bb, onesb, zbuf, psp, csp, sem1, sem2):
    c = lax.axis_index("c")
    s = lax.axis_index("s")
    tid = s * NC + c
    _zero_rows(zbuf, NGP // NS)
    pltpu.sync_copy(zbuf, psp.at[pl.ds(s * (NGP // NS), NGP // NS)])
    pltpu.sync_copy(zbuf, csp.at[pl.ds(s * (NGP // NS), NGP // NS)])
    ones = jnp.ones((L,), jnp.float32)

    def obody(i, _):
        onesb[i, :] = ones
        return 0

    lax.fori_loop(0, CHUNK, obody, 0)
    plsc.subcore_barrier()

    def chunk_body(i, _):
        base = tid * (_P_CHUNKS * CHUNK) + i * CHUNK
        a1 = pltpu.async_copy(h_hbm.at[pl.ds(base, CHUNK)], hb, sem1)
        a2 = pltpu.async_copy(b_hbm.at[pl.ds(base, CHUNK)], bb, sem2)
        a1.wait()
        a2.wait()
        pltpu.sync_copy(hb, psp.at[bb], add=True)
        pltpu.sync_copy(onesb, csp.at[bb], add=True)
        return 0

    lax.fori_loop(0, _P_CHUNKS, chunk_body, 0)
    plsc.subcore_barrier()

    @pl.when(s == 0)
    def _():
        pltpu.sync_copy(psp, pool_hbm.at[c])
        pltpu.sync_copy(csp, cnt_hbm.at[c])


_pool = functools.partial(
    pl.kernel,
    out_type=(jax.ShapeDtypeStruct((NC, NGP, 16), jnp.float32),
              jax.ShapeDtypeStruct((NC, NGP, 16), jnp.float32)),
    mesh=_mesh,
    compiler_params=_sc_params,
    scratch_types=[
        pltpu.VMEM((CHUNK, 16), jnp.float32),
        pltpu.VMEM((CHUNK,), jnp.int32),
        pltpu.VMEM((CHUNK, 16), jnp.float32),
        pltpu.VMEM((NGP // NS, 16), jnp.float32),
        pltpu.VMEM_SHARED((NGP, 16), jnp.float32),
        pltpu.VMEM_SHARED((NGP, 16), jnp.float32),
        pltpu.SemaphoreType.DMA,
        pltpu.SemaphoreType.DMA,
    ],
)(_pool_body)


NTC = 50176
_BN = 1024


def _full_spec(shape):
    nd = len(shape)
    return pl.BlockSpec(shape, lambda i: (0,) * nd)


def _tc1_body(x_ref, w0_ref, b0_ref, wl_ref, wr_ref, xl_ref, xr_ref):
    xb = x_ref[...]
    ohe = xb[:, :1] @ w0_ref[...] + b0_ref[...][None, :]
    h = jnp.concatenate([ohe, xb[:, 1:]], axis=1)
    xl_ref[...] = h @ wl_ref[...]
    xr_ref[...] = h @ wr_ref[...]


def _tc1(x_p, W0, b0, Wl, Wr):
    C = Wl.shape[1]
    return pl.pallas_call(
        _tc1_body,
        grid=(NTC // _BN,),
        in_specs=[pl.BlockSpec((_BN, 4), lambda i: (i, 0)),
                  _full_spec(W0.shape), _full_spec(b0.shape),
                  _full_spec(Wl.shape), _full_spec(Wr.shape)],
        out_specs=(pl.BlockSpec((_BN, C), lambda i: (i, 0)),
                   pl.BlockSpec((_BN, C), lambda i: (i, 0))),
        out_shape=(jax.ShapeDtypeStruct((NTC, C), jnp.float32),
                   jax.ShapeDtypeStruct((NTC, C), jnp.float32)),
    )(x_p, W0, b0, Wl, Wr)


def _tc2_body(in_ref, b_ref, wl_ref, wr_ref, xl_ref, xr_ref):
    h = jnp.tanh(in_ref[...] + b_ref[...][None, :])
    xl_ref[...] = h @ wl_ref[...]
    xr_ref[...] = h @ wr_ref[...]


def _tc2(h_raw, b, Wl, Wr):
    Cin = Wl.shape[0]
    C = Wl.shape[1]
    return pl.pallas_call(
        _tc2_body,
        grid=(NTC // _BN,),
        in_specs=[pl.BlockSpec((_BN, Cin), lambda i: (i, 0)),
                  _full_spec(b.shape),
                  _full_spec(Wl.shape), _full_spec(Wr.shape)],
        out_specs=(pl.BlockSpec((_BN, C), lambda i: (i, 0)),
                   pl.BlockSpec((_BN, C), lambda i: (i, 0))),
        out_shape=(jax.ShapeDtypeStruct((NTC, C), jnp.float32),
                   jax.ShapeDtypeStruct((NTC, C), jnp.float32)),
    )(h_raw, b, Wl, Wr)


def _tanh_body(in_ref, b_ref, o_ref):
    o_ref[...] = jnp.tanh(in_ref[...] + b_ref[...][None, :])


def _tc_tanh(h_raw, b):
    C = h_raw.shape[1]
    return pl.pallas_call(
        _tanh_body,
        grid=(NTC // _BN,),
        in_specs=[pl.BlockSpec((_BN, C), lambda i: (i, 0)),
                  _full_spec(b.shape)],
        out_specs=pl.BlockSpec((_BN, C), lambda i: (i, 0)),
        out_shape=jax.ShapeDtypeStruct((NTC, C), jnp.float32),
    )(h_raw, b)


def _gat_sparse(meta, xlf, xrf, We, att, C, heads):
    we_rep = jnp.tile(We[0][:, None], (1, 16))
    att_rep = jnp.tile(att.reshape(-1)[:, None], (1, 16))
    w = _attention_weights_sc(meta, xlf, xrf, we_rep, att_rep, C, heads)
    return _aggregate_sc(meta, w, xlf, C)


def _tail_kernel(g_ref, w_ref, b_ref, o_ref):
    o_ref[...] = g_ref[...] @ w_ref[...] + b_ref[...]


def kernel(x, edge_index, edge_attr, batch, W0, b0, Wl1, Wr1, We1, att1, b1,
           Wl2, Wr2, We2, att2, b2, Wl3, Wr3, We3, att3, b3, W4, b4):
    n = x.shape[0]
    src = edge_index[0].astype(jnp.int32)
    dst = edge_index[1].astype(jnp.int32)
    loop_attr = _loop_attr_sc(dst, edge_attr[:, 0])
    ar = jnp.arange(n, dtype=jnp.int32)
    src_a = jnp.concatenate([src, ar])
    dst_a = jnp.concatenate([dst, ar])
    ea_all = jnp.concatenate([edge_attr[:, 0], loop_attr])
    src_p = jnp.zeros((EA_P,), jnp.int32).at[:EA].set(src_a)
    dst_p = jnp.full((EA_P,), N, jnp.int32).at[:EA].set(dst_a)
    ea_p = jnp.zeros((EA_P,), jnp.float32).at[:EA].set(ea_all)
    meta = jnp.stack(
        [src_p, dst_p, jax.lax.bitcast_convert_type(ea_p, jnp.int32),
         jnp.zeros((EA_P,), jnp.int32)], axis=1)

    x_p = jnp.zeros((NTC, 4), jnp.float32).at[:n].set(x)
    xl1, xr1 = _tc1(x_p, W0, b0, Wl1, Wr1)
    out1 = _gat_sparse(meta, xl1, xr1, We1, att1, 256, 8)
    xl2, xr2 = _tc2(out1[:NTC], b1, Wl2, Wr2)
    out2 = _gat_sparse(meta, xl2, xr2, We2, att2, 128, 8)
    Wl3p = jnp.pad(Wl3, ((0, 0), (0, 8)))
    Wr3p = jnp.pad(Wr3, ((0, 0), (0, 8)))
    We3p = jnp.pad(We3, ((0, 0), (0, 8)))
    att3p = jnp.pad(att3, ((0, 0), (0, 8)))
    xl3, xr3 = _tc2(out2[:NTC], b2, Wl3p, Wr3p)
    out3 = _gat_sparse(meta, xl3, xr3, We3p, att3p, 16, 1)
    h3t = _tc_tanh(out3[:NTC], jnp.pad(b3, (0, 8)))

    h3_pp = jnp.zeros((N_PP, 16), jnp.float32).at[:n].set(h3t[:n])
    batch_p = jnp.full((N_PP,), NG, jnp.int32).at[:n].set(batch.astype(jnp.int32))
    pool, cnt = _pool(h3_pp, batch_p)
    g = ((pool[0, :NG, :8] + pool[1, :NG, :8])
         / jnp.maximum(cnt[0, :NG, 0] + cnt[1, :NG, 0], 1.0)[:, None])
    out = pl.pallas_call(
        _tail_kernel,
        out_shape=jax.ShapeDtypeStruct((NG, 1), jnp.float32),
    )(g, W4, b4)
    return out
